# baseline (speedup 1.0000x reference)
import jax
import jax.numpy as jnp
from jax.experimental import pallas as pl
from jax.experimental.pallas import tpu as pltpu

_MN = 8
_OFFS = tuple(o for o in range(-_MN, _MN + 1) if o != 0)
_F32 = jnp.float32
_C = 512
_PARALLEL2 = pltpu.CompilerParams(
    dimension_semantics=("parallel", "parallel"))


def _silu(v):
    return v / (1.0 + jnp.exp(-v))


def _mm(x, W):
    return jax.lax.dot_general(x, W, (((1,), (1,)), ((), ())),
                               preferred_element_type=_F32)


def _ln(v, g, b, eps=1e-5):
    m = jnp.mean(v, axis=-1, keepdims=True)
    var = jnp.mean((v - m) ** 2, axis=-1, keepdims=True)
    return (v - m) * jax.lax.rsqrt(var + eps) * g + b


def _normalize(v, eps=1e-12):
    n = jnp.sqrt(jnp.sum(v * v, axis=-1, keepdims=True))
    return v / jnp.maximum(n, eps)


def _row2(a):
    return jnp.asarray(a, _F32).reshape(1, -1)


def _tlin(wb):
    W, b = wb
    return [jnp.asarray(W, _F32), _row2(b)]


def _full_specs(arrs):
    return [pl.BlockSpec(a.shape, (lambda b, c, n=a.ndim: (0,) * n))
            for a in arrs]


def _pre_call(z_g, z_l, wl, B, L, ZG):
    NW = len(wl)

    def body(*refs):
        zg_ref, zl_ref = refs[0], refs[1]
        w = [r[...] for r in refs[2:2 + NW]]
        seq_ref, x_ref, h_ref = refs[2 + NW:]
        (s1W, s1b, sln1g, sln1b, s2W, s2b, sln2g, sln2b, s3W, s3b,
         ltc1W, ltc1b, ltclng, ltclnb, ltc2W, ltc2b, ltc3W, ltc3b,
         embW, embb) = w

        zg = jnp.broadcast_to(zg_ref[0], (_C, ZG))
        zc = jnp.concatenate([zg, zl_ref[0]], axis=-1)

        s = jax.nn.relu(_ln(_mm(zc, s1W) + s1b, sln1g, sln1b))
        s = jax.nn.relu(_ln(_mm(s, s2W) + s2b, sln2g, sln2b))
        seq_ref[0] = _mm(s, s3W) + s3b

        t = jax.nn.relu(_ln(_mm(zc, ltc1W) + ltc1b, ltclng, ltclnb))
        t = jax.nn.relu(_mm(t, ltc2W) + ltc2b)
        x_ref[0] = _mm(t, ltc3W) + ltc3b
        h_ref[0] = _mm(zc, embW) + embb

    return pl.pallas_call(
        body,
        grid=(B, L // _C),
        in_specs=[pl.BlockSpec((1, 1, ZG), lambda b, c: (b, 0, 0)),
                  pl.BlockSpec((1, _C, z_l.shape[-1]),
                               lambda b, c: (b, c, 0))]
                 + _full_specs(wl),
        out_specs=[pl.BlockSpec((1, _C, 20), lambda b, c: (b, c, 0)),
                   pl.BlockSpec((1, _C, 3), lambda b, c: (b, c, 0)),
                   pl.BlockSpec((1, _C, 128), lambda b, c: (b, c, 0))],
        out_shape=[jax.ShapeDtypeStruct((B, L, 20), _F32),
                   jax.ShapeDtypeStruct((B, L, 3), _F32),
                   jax.ShapeDtypeStruct((B, L, 128), _F32)],
        compiler_params=_PARALLEL2,
    )(z_g, z_l, *wl)


def _layer_call(h_pad, x_pad, wl, B, L):
    NW = len(wl)
    LP = L + 2 * _MN

    def body(*refs):
        h_ref, x_ref = refs[0], refs[1]
        w = [r[...] for r in refs[2:2 + NW]]
        ho_ref, xo_ref = refs[2 + NW:]
        (e1W, e1c, e1b, e2W, e2b, h1W, h1b,
         h2W, h2b, x1W, x1b, x2T, x2b, lng, lnb) = w

        cid = pl.program_id(1)
        base = cid * _C
        hs = h_ref[0, pl.ds(base, _C + 2 * _MN), :]
        xs = x_ref[0, pl.ds(base, _C + 2 * _MN), :]
        h0 = hs[_MN:_MN + _C]
        xsT = xs.T
        x0T = xsT[:, _MN:_MN + _C]

        gl = base + jax.lax.broadcasted_iota(jnp.int32, (1, _C), 1)
        degT = (jnp.minimum(gl, _MN) + jnp.minimum(L - 1 - gl, _MN)
                ).astype(_F32)
        deg_invT = 1.0 / degT

        A = _mm(hs, e1W[:, :128])
        Bm = _mm(hs, e1W[:, 128:256])
        Ac = A[_MN:_MN + _C]
        agg = jnp.zeros((_C, 128), _F32)
        cdT = jnp.zeros((3, _C), _F32)
        for off in _OFFS:
            relT = x0T - xsT[:, _MN + off:_MN + off + _C]
            d2T = jnp.sum(relT * relT, axis=0, keepdims=True)
            validT = ((gl + off >= 0) & (gl + off < L)).astype(_F32)
            pre = Ac + Bm[_MN + off:_MN + off + _C] + d2T.T * e1c + e1b
            m = _silu(pre)
            m = _silu(_mm(m, e2W) + e2b)
            agg = agg + m * validT.T
            wv = _silu(_mm(m, x1W) + x1b)
            wsT = jax.lax.dot_general(x2T[:, 0][None, :], wv,
                                      (((1,), (1,)), ((), ())),
                                      preferred_element_type=_F32) + x2b
            cdT = cdT + (wsT * validT) * relT
        hu = _silu(_mm(h0, h1W[:, :128]) + _mm(agg, h1W[:, 128:]) + h1b)
        hu = _mm(hu, h2W) + h2b
        ho_ref[0] = _ln(h0 + hu, lng, lnb)
        xo_ref[0] = (x0T + cdT * deg_invT).T

    return pl.pallas_call(
        body,
        grid=(B, L // _C),
        in_specs=[pl.BlockSpec((1, LP, 128), lambda b, c: (b, 0, 0)),
                  pl.BlockSpec((1, LP, 3), lambda b, c: (b, 0, 0))]
                 + _full_specs(wl),
        out_specs=[pl.BlockSpec((1, _C, 128), lambda b, c: (b, c, 0)),
                   pl.BlockSpec((1, _C, 3), lambda b, c: (b, c, 0))],
        out_shape=[jax.ShapeDtypeStruct((B, L, 128), _F32),
                   jax.ShapeDtypeStruct((B, L, 3), _F32)],
        compiler_params=_PARALLEL2,
    )(h_pad, x_pad, *wl)


def _post_call(h, x, wl, B, L):
    NW = len(wl)

    def body(*refs):
        h_ref, x_ref = refs[0], refs[1]
        w = [r[...] for r in refs[2:2 + NW]]
        n_ref, ca_ref, c_ref = refs[2 + NW:]
        n1W, n1b, n2W, n2b, c1W, c1b, c2W, c2b = w

        h0 = h_ref[0]
        x0 = x_ref[0]
        npred = _mm(jax.nn.relu(_mm(h0, n1W) + n1b), n2W) + n2b
        cpred = _mm(jax.nn.relu(_mm(h0, c1W) + c1b), c2W) + c2b
        n_len = 1.46 + (jax.nn.sigmoid(npred[:, 3:4]) - 0.5) * 0.1
        c_len = 1.52 + (jax.nn.sigmoid(cpred[:, 3:4]) - 0.5) * 0.1
        n_ref[0] = x0 + _normalize(npred[:, :3]) * n_len
        ca_ref[0] = x0
        c_ref[0] = x0 + _normalize(cpred[:, :3]) * c_len

    return pl.pallas_call(
        body,
        grid=(B, L // _C),
        in_specs=[pl.BlockSpec((1, _C, 128), lambda b, c: (b, c, 0)),
                  pl.BlockSpec((1, _C, 3), lambda b, c: (b, c, 0))]
                 + _full_specs(wl),
        out_specs=[pl.BlockSpec((1, _C, 3), lambda b, c: (b, c, 0)),
                   pl.BlockSpec((1, _C, 3), lambda b, c: (b, c, 0)),
                   pl.BlockSpec((1, _C, 3), lambda b, c: (b, c, 0))],
        out_shape=[jax.ShapeDtypeStruct((B, L, 3), _F32),
                   jax.ShapeDtypeStruct((B, L, 3), _F32),
                   jax.ShapeDtypeStruct((B, L, 3), _F32)],
        compiler_params=_PARALLEL2,
    )(h, x, *wl)


def _pad_rows(a):
    return jnp.pad(a, ((0, 0), (_MN, _MN), (0, 0)))


def kernel(z_g, z_l, params):
    B, L, _ = z_l.shape
    ZG = z_g.shape[-1]
    p = params

    pre_w = (_tlin(p['s1']) + [_row2(p['s_ln1'][0]), _row2(p['s_ln1'][1])]
             + _tlin(p['s2']) + [_row2(p['s_ln2'][0]), _row2(p['s_ln2'][1])]
             + _tlin(p['s3'])
             + _tlin(p['ltc1'])
             + [_row2(p['ltc_ln'][0]), _row2(p['ltc_ln'][1])]
             + _tlin(p['ltc2']) + _tlin(p['ltc3'])
             + _tlin(p['emb']))
    layer_w = []
    for lp in p['layers']:
        e1W = jnp.asarray(lp['e1'][0], _F32)
        layer_w.append([e1W, _row2(e1W[:, 256]), _row2(lp['e1'][1])]
                       + _tlin(lp['e2']) + _tlin(lp['h1'])
                       + _tlin(lp['h2']) + _tlin(lp['x1'])
                       + [jnp.asarray(lp['x2'][0], _F32).T,
                          _row2(lp['x2'][1])]
                       + [_row2(lp['ln'][0]), _row2(lp['ln'][1])])
    post_w = (_tlin(p['n1']) + _tlin(p['n2'])
              + _tlin(p['c1']) + _tlin(p['c2']))

    zg3 = jnp.asarray(z_g, _F32).reshape(B, 1, ZG)
    zl3 = jnp.asarray(z_l, _F32)

    seq, x, h = _pre_call(zg3, zl3, pre_w, B, L, ZG)
    for wl in layer_w:
        h, x = _layer_call(_pad_rows(h), _pad_rows(x), wl, B, L)
    n, ca, c = _post_call(h, x, post_w, B, L)
    return (n, ca, c, seq)


# speedup vs baseline: 1.2542x; 1.2542x over previous
import jax
import jax.numpy as jnp
from jax.experimental import pallas as pl
from jax.experimental.pallas import tpu as pltpu

_MN = 8
_OFFS = tuple(o for o in range(-_MN, _MN + 1) if o != 0)
_F32 = jnp.float32
_C = 512
_PARALLEL2 = pltpu.CompilerParams(
    dimension_semantics=("parallel", "parallel"))


def _silu(v):
    return v / (1.0 + jnp.exp(-v))


def _mm(x, W):
    return jax.lax.dot_general(x, W, (((1,), (1,)), ((), ())),
                               preferred_element_type=_F32)


def _ln(v, g, b, eps=1e-5):
    m = jnp.mean(v, axis=-1, keepdims=True)
    var = jnp.mean((v - m) ** 2, axis=-1, keepdims=True)
    return (v - m) / jnp.sqrt(var + eps) * g + b


def _normalize(v, eps=1e-12):
    n = jnp.sqrt(jnp.sum(v * v, axis=-1, keepdims=True))
    return v / jnp.maximum(n, eps)


def _row2(a):
    return jnp.asarray(a, _F32).reshape(1, -1)


def _tlin(wb):
    W, b = wb
    return [jnp.asarray(W, _F32), _row2(b)]


def _full_specs(arrs):
    return [pl.BlockSpec(a.shape, (lambda b, c, n=a.ndim: (0,) * n))
            for a in arrs]


def _pre_call(z_g, z_l, wl, B, L, ZG):
    NW = len(wl)

    def body(*refs):
        zg_ref, zl_ref = refs[0], refs[1]
        w = [r[...] for r in refs[2:2 + NW]]
        seq_ref, x_ref, h_ref = refs[2 + NW:]
        (s1W, s1b, sln1g, sln1b, s2W, s2b, sln2g, sln2b, s3W, s3b,
         ltc1W, ltc1b, ltclng, ltclnb, ltc2W, ltc2b, ltc3W, ltc3b,
         embW, embb) = w

        zg = jnp.broadcast_to(zg_ref[0], (_C, ZG))
        zc = jnp.concatenate([zg, zl_ref[0]], axis=-1)

        s = jax.nn.relu(_ln(_mm(zc, s1W) + s1b, sln1g, sln1b))
        s = jax.nn.relu(_ln(_mm(s, s2W) + s2b, sln2g, sln2b))
        seq_ref[0] = _mm(s, s3W) + s3b

        t = jax.nn.relu(_ln(_mm(zc, ltc1W) + ltc1b, ltclng, ltclnb))
        t = jax.nn.relu(_mm(t, ltc2W) + ltc2b)
        x_ref[0] = _mm(t, ltc3W) + ltc3b
        h_ref[0] = _mm(zc, embW) + embb

    return pl.pallas_call(
        body,
        grid=(B, L // _C),
        in_specs=[pl.BlockSpec((1, 1, ZG), lambda b, c: (b, 0, 0)),
                  pl.BlockSpec((1, _C, z_l.shape[-1]),
                               lambda b, c: (b, c, 0))]
                 + _full_specs(wl),
        out_specs=[pl.BlockSpec((1, _C, 20), lambda b, c: (b, c, 0)),
                   pl.BlockSpec((1, _C, 3), lambda b, c: (b, c, 0)),
                   pl.BlockSpec((1, _C, 128), lambda b, c: (b, c, 0))],
        out_shape=[jax.ShapeDtypeStruct((B, L, 20), _F32),
                   jax.ShapeDtypeStruct((B, L, 3), _F32),
                   jax.ShapeDtypeStruct((B, L, 128), _F32)],
        compiler_params=_PARALLEL2,
    )(z_g, z_l, *wl)


def _layer_call(h_pad, x_pad, wl, B, L):
    NW = len(wl)
    LP = L + 2 * _MN

    def body(*refs):
        h_ref, x_ref = refs[0], refs[1]
        w = [r[...] for r in refs[2:2 + NW]]
        ho_ref, xo_ref = refs[2 + NW:]
        (e1W, e1c, e1b, e2W, e2b, h1W, h1b,
         h2W, h2b, x1W, x1b, x2T, x2b, lng, lnb) = w

        cid = pl.program_id(1)
        base = cid * _C
        hs = h_ref[0, pl.ds(base, _C + 2 * _MN), :]
        xs = x_ref[0, pl.ds(base, _C + 2 * _MN), :]
        h0 = hs[_MN:_MN + _C]
        x0 = xs[_MN:_MN + _C]

        gi = base + jax.lax.broadcasted_iota(jnp.int32, (_C, 1), 0)
        deg = (jnp.minimum(gi, _MN) + jnp.minimum(L - 1 - gi, _MN)
               ).astype(_F32)

        A = _mm(hs, e1W[:, :128])
        Bm = _mm(hs, e1W[:, 128:256])
        Ac = A[_MN:_MN + _C]
        agg = jnp.zeros((_C, 128), _F32)
        cd = jnp.zeros((_C, 3), _F32)
        for off in _OFFS:
            rel = x0 - xs[_MN + off:_MN + off + _C]
            d2 = jnp.sum(rel * rel, axis=-1, keepdims=True)
            pre = Ac + Bm[_MN + off:_MN + off + _C] + d2 * e1c + e1b
            m = _silu(pre)
            m = _silu(_mm(m, e2W) + e2b)
            valid = ((gi + off >= 0) & (gi + off < L)).astype(_F32)
            agg = agg + m * valid
            wv = _silu(_mm(m, x1W) + x1b)
            ws = wv @ x2T + x2b
            cd = cd + (ws * valid) * rel
        hu = _silu(_mm(h0, h1W[:, :128]) + _mm(agg, h1W[:, 128:]) + h1b)
        hu = _mm(hu, h2W) + h2b
        ho_ref[0] = _ln(h0 + hu, lng, lnb)
        xo_ref[0] = x0 + cd / deg

    return pl.pallas_call(
        body,
        grid=(B, L // _C),
        in_specs=[pl.BlockSpec((1, LP, 128), lambda b, c: (b, 0, 0)),
                  pl.BlockSpec((1, LP, 3), lambda b, c: (b, 0, 0))]
                 + _full_specs(wl),
        out_specs=[pl.BlockSpec((1, _C, 128), lambda b, c: (b, c, 0)),
                   pl.BlockSpec((1, _C, 3), lambda b, c: (b, c, 0))],
        out_shape=[jax.ShapeDtypeStruct((B, L, 128), _F32),
                   jax.ShapeDtypeStruct((B, L, 3), _F32)],
        compiler_params=_PARALLEL2,
    )(h_pad, x_pad, *wl)


def _post_call(h, x, wl, B, L):
    NW = len(wl)

    def body(*refs):
        h_ref, x_ref = refs[0], refs[1]
        w = [r[...] for r in refs[2:2 + NW]]
        n_ref, ca_ref, c_ref = refs[2 + NW:]
        n1W, n1b, n2W, n2b, c1W, c1b, c2W, c2b = w

        h0 = h_ref[0]
        x0 = x_ref[0]
        npred = _mm(jax.nn.relu(_mm(h0, n1W) + n1b), n2W) + n2b
        cpred = _mm(jax.nn.relu(_mm(h0, c1W) + c1b), c2W) + c2b
        n_len = 1.46 + (jax.nn.sigmoid(npred[:, 3:4]) - 0.5) * 0.1
        c_len = 1.52 + (jax.nn.sigmoid(cpred[:, 3:4]) - 0.5) * 0.1
        n_ref[0] = x0 + _normalize(npred[:, :3]) * n_len
        ca_ref[0] = x0
        c_ref[0] = x0 + _normalize(cpred[:, :3]) * c_len

    return pl.pallas_call(
        body,
        grid=(B, L // _C),
        in_specs=[pl.BlockSpec((1, _C, 128), lambda b, c: (b, c, 0)),
                  pl.BlockSpec((1, _C, 3), lambda b, c: (b, c, 0))]
                 + _full_specs(wl),
        out_specs=[pl.BlockSpec((1, _C, 3), lambda b, c: (b, c, 0)),
                   pl.BlockSpec((1, _C, 3), lambda b, c: (b, c, 0)),
                   pl.BlockSpec((1, _C, 3), lambda b, c: (b, c, 0))],
        out_shape=[jax.ShapeDtypeStruct((B, L, 3), _F32),
                   jax.ShapeDtypeStruct((B, L, 3), _F32),
                   jax.ShapeDtypeStruct((B, L, 3), _F32)],
        compiler_params=_PARALLEL2,
    )(h, x, *wl)


def _pad_rows(a):
    return jnp.pad(a, ((0, 0), (_MN, _MN), (0, 0)))


def kernel(z_g, z_l, params):
    B, L, _ = z_l.shape
    ZG = z_g.shape[-1]
    p = params

    pre_w = (_tlin(p['s1']) + [_row2(p['s_ln1'][0]), _row2(p['s_ln1'][1])]
             + _tlin(p['s2']) + [_row2(p['s_ln2'][0]), _row2(p['s_ln2'][1])]
             + _tlin(p['s3'])
             + _tlin(p['ltc1'])
             + [_row2(p['ltc_ln'][0]), _row2(p['ltc_ln'][1])]
             + _tlin(p['ltc2']) + _tlin(p['ltc3'])
             + _tlin(p['emb']))
    layer_w = []
    for lp in p['layers']:
        e1W = jnp.asarray(lp['e1'][0], _F32)
        layer_w.append([e1W, _row2(e1W[:, 256]), _row2(lp['e1'][1])]
                       + _tlin(lp['e2']) + _tlin(lp['h1'])
                       + _tlin(lp['h2']) + _tlin(lp['x1'])
                       + [jnp.asarray(lp['x2'][0], _F32).T,
                          _row2(lp['x2'][1])]
                       + [_row2(lp['ln'][0]), _row2(lp['ln'][1])])
    post_w = (_tlin(p['n1']) + _tlin(p['n2'])
              + _tlin(p['c1']) + _tlin(p['c2']))

    zg3 = jnp.asarray(z_g, _F32).reshape(B, 1, ZG)
    zl3 = jnp.asarray(z_l, _F32)

    seq, x, h = _pre_call(zg3, zl3, pre_w, B, L, ZG)
    for wl in layer_w:
        h, x = _layer_call(_pad_rows(h), _pad_rows(x), wl, B, L)
    n, ca, c = _post_call(h, x, post_w, B, L)
    return (n, ca, c, seq)


# speedup vs baseline: 1.2805x; 1.0210x over previous
import jax
import jax.numpy as jnp
from jax.experimental import pallas as pl
from jax.experimental.pallas import tpu as pltpu

_MN = 8
_OFFS = tuple(o for o in range(-_MN, _MN + 1) if o != 0)
_F32 = jnp.float32
_C = 512
_PARALLEL2 = pltpu.CompilerParams(
    dimension_semantics=("parallel", "parallel"))


def _silu(v):
    return v / (1.0 + jnp.exp(-v))


def _mm(x, W):
    return jax.lax.dot_general(x, W, (((1,), (1,)), ((), ())),
                               preferred_element_type=_F32)


def _ln(v, g, b, eps=1e-5):
    m = jnp.mean(v, axis=-1, keepdims=True)
    var = jnp.mean((v - m) ** 2, axis=-1, keepdims=True)
    return (v - m) / jnp.sqrt(var + eps) * g + b


def _normalize(v, eps=1e-12):
    n = jnp.sqrt(jnp.sum(v * v, axis=-1, keepdims=True))
    return v / jnp.maximum(n, eps)


def _row2(a):
    return jnp.asarray(a, _F32).reshape(1, -1)


def _tlin(wb):
    W, b = wb
    return [jnp.asarray(W, _F32), _row2(b)]


def _full_specs(arrs):
    return [pl.BlockSpec(a.shape, (lambda b, c, n=a.ndim: (0,) * n))
            for a in arrs]


def _pre_call(z_g, z_l, wl, B, L, ZG):
    NW = len(wl)

    def body(*refs):
        zg_ref, zl_ref = refs[0], refs[1]
        w = [r[...] for r in refs[2:2 + NW]]
        seq_ref, x_ref, h_ref = refs[2 + NW:]
        (s1W, s1b, sln1g, sln1b, s2W, s2b, sln2g, sln2b, s3W, s3b,
         ltc1W, ltc1b, ltclng, ltclnb, ltc2W, ltc2b, ltc3W, ltc3b,
         embW, embb) = w

        zg = jnp.broadcast_to(zg_ref[0], (_C, ZG))
        zc = jnp.concatenate([zg, zl_ref[0]], axis=-1)

        s = jax.nn.relu(_ln(_mm(zc, s1W) + s1b, sln1g, sln1b))
        s = jax.nn.relu(_ln(_mm(s, s2W) + s2b, sln2g, sln2b))
        seq_ref[0] = _mm(s, s3W) + s3b

        t = jax.nn.relu(_ln(_mm(zc, ltc1W) + ltc1b, ltclng, ltclnb))
        t = jax.nn.relu(_mm(t, ltc2W) + ltc2b)
        x_ref[0] = _mm(t, ltc3W) + ltc3b
        h_ref[0] = _mm(zc, embW) + embb

    return pl.pallas_call(
        body,
        grid=(B, L // _C),
        in_specs=[pl.BlockSpec((1, 1, ZG), lambda b, c: (b, 0, 0)),
                  pl.BlockSpec((1, _C, z_l.shape[-1]),
                               lambda b, c: (b, c, 0))]
                 + _full_specs(wl),
        out_specs=[pl.BlockSpec((1, _C, 20), lambda b, c: (b, c, 0)),
                   pl.BlockSpec((1, _C, 3), lambda b, c: (b, c, 0)),
                   pl.BlockSpec((1, _C, 128), lambda b, c: (b, c, 0))],
        out_shape=[jax.ShapeDtypeStruct((B, L, 20), _F32),
                   jax.ShapeDtypeStruct((B, L, 3), _F32),
                   jax.ShapeDtypeStruct((B, L, 128), _F32)],
        compiler_params=_PARALLEL2,
    )(z_g, z_l, *wl)


def _layer_call(h_pad, x_pad, wl, B, L):
    NW = len(wl)
    LP = L + 2 * _MN

    def body(*refs):
        h_ref, x_ref = refs[0], refs[1]
        w = [r[...] for r in refs[2:2 + NW]]
        ho_ref, xo_ref = refs[2 + NW:]
        (e1W, e1c, e1b, e2W, e2b, h1W, h1b,
         h2W, h2b, x1W, x1b, x2T, x2b, lng, lnb) = w

        cid = pl.program_id(1)
        base = cid * _C
        hs = h_ref[0, pl.ds(base, _C + 2 * _MN), :]
        xs = x_ref[0, pl.ds(base, _C + 2 * _MN), :]
        h0 = hs[_MN:_MN + _C]
        x0 = xs[_MN:_MN + _C]

        gi = base + jax.lax.broadcasted_iota(jnp.int32, (_C, 1), 0)
        deg = (jnp.minimum(gi, _MN) + jnp.minimum(L - 1 - gi, _MN)
               ).astype(_F32)

        AB = _mm(hs, e1W)
        A = AB[:, :128]
        Bm = AB[:, 128:]
        Ac = A[_MN:_MN + _C]
        agg = jnp.zeros((_C, 128), _F32)
        cd = jnp.zeros((_C, 3), _F32)
        for off in _OFFS:
            rel = x0 - xs[_MN + off:_MN + off + _C]
            d2 = jnp.sum(rel * rel, axis=-1, keepdims=True)
            pre = Ac + Bm[_MN + off:_MN + off + _C] + d2 * e1c + e1b
            m = _silu(pre)
            m = _silu(_mm(m, e2W) + e2b)
            valid = ((gi + off >= 0) & (gi + off < L)).astype(_F32)
            agg = agg + m * valid
            wv = _silu(_mm(m, x1W) + x1b)
            ws = wv @ x2T + x2b
            cd = cd + (ws * valid) * rel
        hu = _silu(_mm(jnp.concatenate([h0, agg], axis=-1), h1W) + h1b)
        hu = _mm(hu, h2W) + h2b
        ho_ref[0] = _ln(h0 + hu, lng, lnb)
        xo_ref[0] = x0 + cd / deg

    return pl.pallas_call(
        body,
        grid=(B, L // _C),
        in_specs=[pl.BlockSpec((1, LP, 128), lambda b, c: (b, 0, 0)),
                  pl.BlockSpec((1, LP, 3), lambda b, c: (b, 0, 0))]
                 + _full_specs(wl),
        out_specs=[pl.BlockSpec((1, _C, 128), lambda b, c: (b, c, 0)),
                   pl.BlockSpec((1, _C, 3), lambda b, c: (b, c, 0))],
        out_shape=[jax.ShapeDtypeStruct((B, L, 128), _F32),
                   jax.ShapeDtypeStruct((B, L, 3), _F32)],
        compiler_params=_PARALLEL2,
    )(h_pad, x_pad, *wl)


def _post_call(h, x, wl, B, L):
    NW = len(wl)

    def body(*refs):
        h_ref, x_ref = refs[0], refs[1]
        w = [r[...] for r in refs[2:2 + NW]]
        n_ref, ca_ref, c_ref = refs[2 + NW:]
        n1W, n1b, n2W, n2b, c1W, c1b, c2W, c2b = w

        h0 = h_ref[0]
        x0 = x_ref[0]
        npred = _mm(jax.nn.relu(_mm(h0, n1W) + n1b), n2W) + n2b
        cpred = _mm(jax.nn.relu(_mm(h0, c1W) + c1b), c2W) + c2b
        n_len = 1.46 + (jax.nn.sigmoid(npred[:, 3:4]) - 0.5) * 0.1
        c_len = 1.52 + (jax.nn.sigmoid(cpred[:, 3:4]) - 0.5) * 0.1
        n_ref[0] = x0 + _normalize(npred[:, :3]) * n_len
        ca_ref[0] = x0
        c_ref[0] = x0 + _normalize(cpred[:, :3]) * c_len

    return pl.pallas_call(
        body,
        grid=(B, L // _C),
        in_specs=[pl.BlockSpec((1, _C, 128), lambda b, c: (b, c, 0)),
                  pl.BlockSpec((1, _C, 3), lambda b, c: (b, c, 0))]
                 + _full_specs(wl),
        out_specs=[pl.BlockSpec((1, _C, 3), lambda b, c: (b, c, 0)),
                   pl.BlockSpec((1, _C, 3), lambda b, c: (b, c, 0)),
                   pl.BlockSpec((1, _C, 3), lambda b, c: (b, c, 0))],
        out_shape=[jax.ShapeDtypeStruct((B, L, 3), _F32),
                   jax.ShapeDtypeStruct((B, L, 3), _F32),
                   jax.ShapeDtypeStruct((B, L, 3), _F32)],
        compiler_params=_PARALLEL2,
    )(h, x, *wl)


def _pad_rows(a):
    return jnp.pad(a, ((0, 0), (_MN, _MN), (0, 0)))


def kernel(z_g, z_l, params):
    B, L, _ = z_l.shape
    ZG = z_g.shape[-1]
    p = params

    pre_w = (_tlin(p['s1']) + [_row2(p['s_ln1'][0]), _row2(p['s_ln1'][1])]
             + _tlin(p['s2']) + [_row2(p['s_ln2'][0]), _row2(p['s_ln2'][1])]
             + _tlin(p['s3'])
             + _tlin(p['ltc1'])
             + [_row2(p['ltc_ln'][0]), _row2(p['ltc_ln'][1])]
             + _tlin(p['ltc2']) + _tlin(p['ltc3'])
             + _tlin(p['emb']))
    layer_w = []
    for lp in p['layers']:
        e1W = jnp.asarray(lp['e1'][0], _F32)
        e1ab = jnp.concatenate([e1W[:, :128], e1W[:, 128:256]], axis=0)
        layer_w.append([e1ab, _row2(e1W[:, 256]), _row2(lp['e1'][1])]
                       + _tlin(lp['e2']) + _tlin(lp['h1'])
                       + _tlin(lp['h2']) + _tlin(lp['x1'])
                       + [jnp.asarray(lp['x2'][0], _F32).T,
                          _row2(lp['x2'][1])]
                       + [_row2(lp['ln'][0]), _row2(lp['ln'][1])])
    post_w = (_tlin(p['n1']) + _tlin(p['n2'])
              + _tlin(p['c1']) + _tlin(p['c2']))

    zg3 = jnp.asarray(z_g, _F32).reshape(B, 1, ZG)
    zl3 = jnp.asarray(z_l, _F32)

    seq, x, h = _pre_call(zg3, zl3, pre_w, B, L, ZG)
    for wl in layer_w:
        h, x = _layer_call(_pad_rows(h), _pad_rows(x), wl, B, L)
    n, ca, c = _post_call(h, x, post_w, B, L)
    return (n, ca, c, seq)


# speedup vs baseline: 1.3333x; 1.0412x over previous
import jax
import jax.numpy as jnp
from jax.experimental import pallas as pl
from jax.experimental.pallas import tpu as pltpu

_MN = 8
_OFFS = tuple(o for o in range(-_MN, _MN + 1) if o != 0)
_F32 = jnp.float32
_C = 1024
_PARALLEL2 = pltpu.CompilerParams(
    dimension_semantics=("parallel", "parallel"))


def _silu(v):
    return v / (1.0 + jnp.exp(-v))


def _mm(x, W):
    return jax.lax.dot_general(x, W, (((1,), (1,)), ((), ())),
                               preferred_element_type=_F32)


def _ln(v, g, b, eps=1e-5):
    m = jnp.mean(v, axis=-1, keepdims=True)
    var = jnp.mean((v - m) ** 2, axis=-1, keepdims=True)
    return (v - m) / jnp.sqrt(var + eps) * g + b


def _normalize(v, eps=1e-12):
    n = jnp.sqrt(jnp.sum(v * v, axis=-1, keepdims=True))
    return v / jnp.maximum(n, eps)


def _row2(a):
    return jnp.asarray(a, _F32).reshape(1, -1)


def _tlin(wb):
    W, b = wb
    return [jnp.asarray(W, _F32), _row2(b)]


def _full_specs(arrs):
    return [pl.BlockSpec(a.shape, (lambda b, c, n=a.ndim: (0,) * n))
            for a in arrs]


def _pre_call(z_g, z_l, wl, B, L, ZG):
    NW = len(wl)

    def body(*refs):
        zg_ref, zl_ref = refs[0], refs[1]
        w = [r[...] for r in refs[2:2 + NW]]
        seq_ref, x_ref, h_ref = refs[2 + NW:]
        (s1W, s1b, sln1g, sln1b, s2W, s2b, sln2g, sln2b, s3W, s3b,
         ltc1W, ltc1b, ltclng, ltclnb, ltc2W, ltc2b, ltc3W, ltc3b,
         embW, embb) = w

        zg = jnp.broadcast_to(zg_ref[0], (_C, ZG))
        zc = jnp.concatenate([zg, zl_ref[0]], axis=-1)

        s = jax.nn.relu(_ln(_mm(zc, s1W) + s1b, sln1g, sln1b))
        s = jax.nn.relu(_ln(_mm(s, s2W) + s2b, sln2g, sln2b))
        seq_ref[0] = _mm(s, s3W) + s3b

        t = jax.nn.relu(_ln(_mm(zc, ltc1W) + ltc1b, ltclng, ltclnb))
        t = jax.nn.relu(_mm(t, ltc2W) + ltc2b)
        x_ref[0] = _mm(t, ltc3W) + ltc3b
        h_ref[0] = _mm(zc, embW) + embb

    return pl.pallas_call(
        body,
        grid=(B, L // _C),
        in_specs=[pl.BlockSpec((1, 1, ZG), lambda b, c: (b, 0, 0)),
                  pl.BlockSpec((1, _C, z_l.shape[-1]),
                               lambda b, c: (b, c, 0))]
                 + _full_specs(wl),
        out_specs=[pl.BlockSpec((1, _C, 20), lambda b, c: (b, c, 0)),
                   pl.BlockSpec((1, _C, 3), lambda b, c: (b, c, 0)),
                   pl.BlockSpec((1, _C, 128), lambda b, c: (b, c, 0))],
        out_shape=[jax.ShapeDtypeStruct((B, L, 20), _F32),
                   jax.ShapeDtypeStruct((B, L, 3), _F32),
                   jax.ShapeDtypeStruct((B, L, 128), _F32)],
        compiler_params=_PARALLEL2,
    )(z_g, z_l, *wl)


def _layer_call(h_pad, x_pad, wl, B, L):
    NW = len(wl)
    LP = L + 2 * _MN

    def body(*refs):
        h_ref, x_ref = refs[0], refs[1]
        w = [r[...] for r in refs[2:2 + NW]]
        ho_ref, xo_ref = refs[2 + NW:]
        (e1W, e1c, e1b, e2W, e2b, h1W, h1b,
         h2W, h2b, x1W, x1b, x2T, x2b, lng, lnb) = w

        cid = pl.program_id(1)
        base = cid * _C
        hs = h_ref[0, pl.ds(base, _C + 2 * _MN), :]
        xs = x_ref[0, pl.ds(base, _C + 2 * _MN), :]
        h0 = hs[_MN:_MN + _C]
        x0 = xs[_MN:_MN + _C]

        gi = base + jax.lax.broadcasted_iota(jnp.int32, (_C, 1), 0)
        deg = (jnp.minimum(gi, _MN) + jnp.minimum(L - 1 - gi, _MN)
               ).astype(_F32)

        AB = _mm(hs, e1W)
        A = AB[:, :128]
        Bm = AB[:, 128:]
        Ac = A[_MN:_MN + _C]
        agg = jnp.zeros((_C, 128), _F32)
        cd = jnp.zeros((_C, 3), _F32)
        for off in _OFFS:
            rel = x0 - xs[_MN + off:_MN + off + _C]
            d2 = jnp.sum(rel * rel, axis=-1, keepdims=True)
            pre = Ac + Bm[_MN + off:_MN + off + _C] + d2 * e1c + e1b
            m = _silu(pre)
            m = _silu(_mm(m, e2W) + e2b)
            valid = ((gi + off >= 0) & (gi + off < L)).astype(_F32)
            agg = agg + m * valid
            wv = _silu(_mm(m, x1W) + x1b)
            ws = wv @ x2T + x2b
            cd = cd + (ws * valid) * rel
        hu = _silu(_mm(jnp.concatenate([h0, agg], axis=-1), h1W) + h1b)
        hu = _mm(hu, h2W) + h2b
        ho_ref[0] = _ln(h0 + hu, lng, lnb)
        xo_ref[0] = x0 + cd / deg

    return pl.pallas_call(
        body,
        grid=(B, L // _C),
        in_specs=[pl.BlockSpec((1, LP, 128), lambda b, c: (b, 0, 0)),
                  pl.BlockSpec((1, LP, 3), lambda b, c: (b, 0, 0))]
                 + _full_specs(wl),
        out_specs=[pl.BlockSpec((1, _C, 128), lambda b, c: (b, c, 0)),
                   pl.BlockSpec((1, _C, 3), lambda b, c: (b, c, 0))],
        out_shape=[jax.ShapeDtypeStruct((B, L, 128), _F32),
                   jax.ShapeDtypeStruct((B, L, 3), _F32)],
        compiler_params=_PARALLEL2,
    )(h_pad, x_pad, *wl)


def _post_call(h, x, wl, B, L):
    NW = len(wl)

    def body(*refs):
        h_ref, x_ref = refs[0], refs[1]
        w = [r[...] for r in refs[2:2 + NW]]
        n_ref, ca_ref, c_ref = refs[2 + NW:]
        n1W, n1b, n2W, n2b, c1W, c1b, c2W, c2b = w

        h0 = h_ref[0]
        x0 = x_ref[0]
        npred = _mm(jax.nn.relu(_mm(h0, n1W) + n1b), n2W) + n2b
        cpred = _mm(jax.nn.relu(_mm(h0, c1W) + c1b), c2W) + c2b
        n_len = 1.46 + (jax.nn.sigmoid(npred[:, 3:4]) - 0.5) * 0.1
        c_len = 1.52 + (jax.nn.sigmoid(cpred[:, 3:4]) - 0.5) * 0.1
        n_ref[0] = x0 + _normalize(npred[:, :3]) * n_len
        ca_ref[0] = x0
        c_ref[0] = x0 + _normalize(cpred[:, :3]) * c_len

    return pl.pallas_call(
        body,
        grid=(B, L // _C),
        in_specs=[pl.BlockSpec((1, _C, 128), lambda b, c: (b, c, 0)),
                  pl.BlockSpec((1, _C, 3), lambda b, c: (b, c, 0))]
                 + _full_specs(wl),
        out_specs=[pl.BlockSpec((1, _C, 3), lambda b, c: (b, c, 0)),
                   pl.BlockSpec((1, _C, 3), lambda b, c: (b, c, 0)),
                   pl.BlockSpec((1, _C, 3), lambda b, c: (b, c, 0))],
        out_shape=[jax.ShapeDtypeStruct((B, L, 3), _F32),
                   jax.ShapeDtypeStruct((B, L, 3), _F32),
                   jax.ShapeDtypeStruct((B, L, 3), _F32)],
        compiler_params=_PARALLEL2,
    )(h, x, *wl)


def _pad_rows(a):
    return jnp.pad(a, ((0, 0), (_MN, _MN), (0, 0)))


def kernel(z_g, z_l, params):
    B, L, _ = z_l.shape
    ZG = z_g.shape[-1]
    p = params

    pre_w = (_tlin(p['s1']) + [_row2(p['s_ln1'][0]), _row2(p['s_ln1'][1])]
             + _tlin(p['s2']) + [_row2(p['s_ln2'][0]), _row2(p['s_ln2'][1])]
             + _tlin(p['s3'])
             + _tlin(p['ltc1'])
             + [_row2(p['ltc_ln'][0]), _row2(p['ltc_ln'][1])]
             + _tlin(p['ltc2']) + _tlin(p['ltc3'])
             + _tlin(p['emb']))
    layer_w = []
    for lp in p['layers']:
        e1W = jnp.asarray(lp['e1'][0], _F32)
        e1ab = jnp.concatenate([e1W[:, :128], e1W[:, 128:256]], axis=0)
        layer_w.append([e1ab, _row2(e1W[:, 256]), _row2(lp['e1'][1])]
                       + _tlin(lp['e2']) + _tlin(lp['h1'])
                       + _tlin(lp['h2']) + _tlin(lp['x1'])
                       + [jnp.asarray(lp['x2'][0], _F32).T,
                          _row2(lp['x2'][1])]
                       + [_row2(lp['ln'][0]), _row2(lp['ln'][1])])
    post_w = (_tlin(p['n1']) + _tlin(p['n2'])
              + _tlin(p['c1']) + _tlin(p['c2']))

    zg3 = jnp.asarray(z_g, _F32).reshape(B, 1, ZG)
    zl3 = jnp.asarray(z_l, _F32)

    seq, x, h = _pre_call(zg3, zl3, pre_w, B, L, ZG)
    for wl in layer_w:
        h, x = _layer_call(_pad_rows(h), _pad_rows(x), wl, B, L)
    n, ca, c = _post_call(h, x, post_w, B, L)
    return (n, ca, c, seq)
